# Initial kernel scaffold; baseline (speedup 1.0000x reference)
import jax
import jax.numpy as jnp
from jax.experimental import pallas as pl


def kernel(af, bf, edge_index, enc1_w, enc1_b, bn1_g, bn1_b, enc2_w, enc2_b, bn2_g, bn2_b, ef_w, ef_b, gru_wi, gru_wh, gru_bi, gru_bh):
    raise NotImplementedError("write your pallas kernel here")


# speedup vs baseline: 1.2157x; 1.2157x over previous
import functools

import jax
import jax.numpy as jnp
from jax import lax
from jax.experimental import pallas as pl
from jax.experimental.pallas import tpu as pltpu
from jax.experimental.pallas import tpu_sc as plsc

HID = 32
NC, NS = 2, 16
NW = NC * NS
CHUNK = 128


def _encoder_body(af_ref, w1t_ref, p1_ref, w2t_ref, p2_ref, h_ref):
    def layer(x, wt_ref, p_ref):
        y = jnp.dot(x, wt_ref[...], preferred_element_type=jnp.float32)
        y = y + p_ref[0:1, :]
        mu = jnp.mean(y, axis=0, keepdims=True)
        d = y - mu
        var = jnp.mean(d * d, axis=0, keepdims=True)
        y = d * (p_ref[1:2, :] * lax.rsqrt(var + 1e-5)) + p_ref[2:3, :]
        return jnp.maximum(y, 0.0)

    h = layer(af_ref[...], w1t_ref, p1_ref)
    h_ref[...] = layer(h, w2t_ref, p2_ref)


def _encoder(af, w1t, p1, w2t, p2):
    n = af.shape[0]
    return pl.pallas_call(
        _encoder_body,
        out_shape=jax.ShapeDtypeStruct((n, HID), jnp.float32),
    )(af, w1t, p1, w2t, p2)


def _gather(h, src3, e_pad, nch):
    mesh = plsc.VectorSubcoreMesh(core_axis_name="c", subcore_axis_name="s")

    @functools.partial(
        pl.kernel,
        mesh=mesh,
        out_type=jax.ShapeDtypeStruct((e_pad, HID), jnp.float32),
        compiler_params=pltpu.CompilerParams(use_tc_tiling_on_sc=False),
        scratch_types=[
            pltpu.VMEM((nch, CHUNK), jnp.int32),
            pltpu.VMEM((CHUNK, HID), jnp.float32),
            pltpu.SemaphoreType.DMA,
        ],
    )
    def k(h_hbm, src_hbm, out_hbm, idx_v, rows_v, sem):
        wid = lax.axis_index("c") * NS + lax.axis_index("s")
        pltpu.sync_copy(src_hbm.at[wid], idx_v)

        def body(j, carry):
            off = (wid * nch + j) * CHUNK
            pltpu.async_copy(h_hbm.at[idx_v.at[j]], rows_v, sem).wait()
            pltpu.sync_copy(rows_v, out_hbm.at[pl.ds(off, CHUNK)])
            return carry

        lax.fori_loop(0, nch, body, 0)

    return k(h, src3)


def _msg_body(hs_ref, bf_ref, ewt_ref, eb_ref, out_ref):
    wb = jnp.dot(bf_ref[...], ewt_ref[...],
                 preferred_element_type=jnp.float32)
    wb = (wb + eb_ref[...]).astype(jnp.bfloat16).astype(jnp.float32)
    hs = hs_ref[...].astype(jnp.bfloat16).astype(jnp.float32)
    acc = wb[:, 0:HID] * hs[:, 0:1]
    for i in range(1, HID):
        acc = acc + wb[:, i * HID:(i + 1) * HID] * hs[:, i:i + 1]
    out_ref[...] = acc


def _msg(h_src, bf_p, ewt, eb, e_pad):
    blk = 1024
    grid = e_pad // blk
    return pl.pallas_call(
        _msg_body,
        grid=(grid,),
        in_specs=[
            pl.BlockSpec((blk, HID), lambda i: (i, 0)),
            pl.BlockSpec((blk, 16), lambda i: (i, 0)),
            pl.BlockSpec((16, HID * HID), lambda i: (0, 0)),
            pl.BlockSpec((1, HID * HID), lambda i: (0, 0)),
        ],
        out_specs=pl.BlockSpec((blk, HID), lambda i: (i, 0)),
        out_shape=jax.ShapeDtypeStruct((e_pad, HID), jnp.float32),
    )(h_src, bf_p, ewt, eb)


def _scatter(msg, dst3, zeros, n_pad, nch):
    mesh = plsc.VectorSubcoreMesh(core_axis_name="c", subcore_axis_name="s")
    rps = n_pad // NS

    @functools.partial(
        pl.kernel,
        mesh=mesh,
        out_type=jax.ShapeDtypeStruct((NC, n_pad, HID), jnp.float32),
        compiler_params=pltpu.CompilerParams(use_tc_tiling_on_sc=False),
        scratch_types=[
            pltpu.VMEM((nch, CHUNK), jnp.int32),
            pltpu.VMEM((CHUNK, HID), jnp.float32),
            pltpu.VMEM_SHARED((n_pad, HID), jnp.float32),
        ],
    )
    def k(msg_hbm, dst_hbm, z_hbm, out_hbm, idx_v, msg_v, agg_sh):
        c = lax.axis_index("c")
        s = lax.axis_index("s")
        wid = c * NS + s
        pltpu.sync_copy(z_hbm.at[pl.ds(s * rps, rps)],
                        agg_sh.at[pl.ds(s * rps, rps)])
        pltpu.sync_copy(dst_hbm.at[wid], idx_v)
        plsc.subcore_barrier()

        def body(j, carry):
            off = (wid * nch + j) * CHUNK
            pltpu.sync_copy(msg_hbm.at[pl.ds(off, CHUNK)], msg_v)
            pltpu.sync_copy(msg_v, agg_sh.at[idx_v.at[j]], add=True)
            return carry

        lax.fori_loop(0, nch, body, 0)
        plsc.subcore_barrier()
        pltpu.sync_copy(agg_sh.at[pl.ds(s * rps, rps)],
                        out_hbm.at[c, pl.ds(s * rps, rps)])

    return k(msg, dst3, zeros)


def _gru_body(h_ref, aggp_ref, wt_ref, bias_ref, out_ref):
    h = h_ref[...]
    agg = aggp_ref[0] + aggp_ref[1]

    def mm(x, i):
        return (jnp.dot(x, wt_ref[i], preferred_element_type=jnp.float32)
                + bias_ref[i:i + 1, :])

    r = jax.nn.sigmoid(mm(agg, 0) + mm(h, 3))
    z = jax.nn.sigmoid(mm(agg, 1) + mm(h, 4))
    nn = jnp.tanh(mm(agg, 2) + r * mm(h, 5))
    out_ref[...] = (1.0 - z) * nn + z * h


def _gru(h, aggp, wt, bias):
    n = h.shape[0]
    blk = 1000
    return pl.pallas_call(
        _gru_body,
        grid=(n // blk,),
        in_specs=[
            pl.BlockSpec((blk, HID), lambda i: (i, 0)),
            pl.BlockSpec((NC, blk, HID), lambda i: (0, i, 0)),
            pl.BlockSpec((6, HID, HID), lambda i: (0, 0, 0)),
            pl.BlockSpec((6, HID), lambda i: (0, 0)),
        ],
        out_specs=pl.BlockSpec((blk, HID), lambda i: (i, 0)),
        out_shape=jax.ShapeDtypeStruct((n, HID), jnp.float32),
    )(h, aggp, wt, bias)


def kernel(af, bf, edge_index, enc1_w, enc1_b, bn1_g, bn1_b, enc2_w, enc2_b,
           bn2_g, bn2_b, ef_w, ef_b, gru_wi, gru_wh, gru_bi, gru_bh):
    n = af.shape[0]
    e = bf.shape[0]
    bond = bf.shape[1]

    nch = -(-e // (NW * CHUNK))
    e_pad = NW * nch * CHUNK
    n_pad = -(-(n + 1) // NS) * NS
    dump = n_pad - 1

    p1 = jnp.stack([enc1_b, bn1_g, bn1_b])
    p2 = jnp.stack([enc2_b, bn2_g, bn2_b])
    h = _encoder(af, enc1_w.T, p1, enc2_w.T, p2)

    src = edge_index[0].astype(jnp.int32)
    dst = edge_index[1].astype(jnp.int32)
    src3 = jnp.pad(src, (0, e_pad - e)).reshape(NW, nch, CHUNK)
    dst3 = jnp.pad(dst, (0, e_pad - e),
                   constant_values=dump).reshape(NW, nch, CHUNK)
    bf_p = jnp.pad(bf, ((0, e_pad - e), (0, 0))).astype(jnp.bfloat16)

    h_src = _gather(h, src3, e_pad, nch)

    ewt = ef_w.T.astype(jnp.bfloat16)
    eb = ef_b.reshape(1, HID * HID)
    msg = _msg(h_src, bf_p, ewt, eb, e_pad)

    zeros = jnp.zeros((n_pad, HID), jnp.float32)
    aggp = _scatter(msg, dst3, zeros, n_pad, nch)

    wt = jnp.stack([
        gru_wi[:HID].T, gru_wi[HID:2 * HID].T, gru_wi[2 * HID:].T,
        gru_wh[:HID].T, gru_wh[HID:2 * HID].T, gru_wh[2 * HID:].T,
    ])
    bias = jnp.stack([
        gru_bi[:HID], gru_bi[HID:2 * HID], gru_bi[2 * HID:],
        gru_bh[:HID], gru_bh[HID:2 * HID], gru_bh[2 * HID:],
    ])
    return _gru(h, aggp, wt, bias)


# speedup vs baseline: 2.7854x; 2.2911x over previous
import functools

import jax
import jax.numpy as jnp
from jax import lax
from jax.experimental import pallas as pl
from jax.experimental.pallas import tpu as pltpu
from jax.experimental.pallas import tpu_sc as plsc

HID = 32
NC, NS = 2, 16
NW = NC * NS
CHUNK = 128


def _encoder_body(af_ref, w1t_ref, p1_ref, w2t_ref, p2_ref, h_ref):
    def layer(x, wt_ref, p_ref):
        y = jnp.dot(x, wt_ref[...], preferred_element_type=jnp.float32)
        y = y + p_ref[0:1, :]
        mu = jnp.mean(y, axis=0, keepdims=True)
        d = y - mu
        var = jnp.mean(d * d, axis=0, keepdims=True)
        y = d * (p_ref[1:2, :] * lax.rsqrt(var + 1e-5)) + p_ref[2:3, :]
        return jnp.maximum(y, 0.0)

    h = layer(af_ref[...], w1t_ref, p1_ref)
    h_ref[...] = layer(h, w2t_ref, p2_ref)


def _encoder(af, w1t, p1, w2t, p2):
    n = af.shape[0]
    return pl.pallas_call(
        _encoder_body,
        out_shape=jax.ShapeDtypeStruct((n, HID), jnp.float32),
    )(af, w1t, p1, w2t, p2)


def _gather(h, src3, e_pad, nch):
    mesh = plsc.VectorSubcoreMesh(core_axis_name="c", subcore_axis_name="s")

    @functools.partial(
        pl.kernel,
        mesh=mesh,
        out_type=jax.ShapeDtypeStruct((e_pad, HID), jnp.float32),
        compiler_params=pltpu.CompilerParams(use_tc_tiling_on_sc=False),
        scratch_types=[
            pltpu.VMEM((nch, CHUNK), jnp.int32),
            pltpu.VMEM((CHUNK, HID), jnp.float32),
            pltpu.SemaphoreType.DMA,
        ],
    )
    def k(h_hbm, src_hbm, out_hbm, idx_v, rows_v, sem):
        wid = lax.axis_index("c") * NS + lax.axis_index("s")
        pltpu.sync_copy(src_hbm.at[wid], idx_v)

        def body(j, carry):
            off = (wid * nch + j) * CHUNK
            pltpu.async_copy(h_hbm.at[idx_v.at[j]], rows_v, sem).wait()
            pltpu.sync_copy(rows_v, out_hbm.at[pl.ds(off, CHUNK)])
            return carry

        lax.fori_loop(0, nch, body, 0)

    return k(h, src3)


def _msg_body(hs_ref, bf_ref, ewt_ref, eb_ref, rep_ref, out_ref):
    wb = jnp.dot(bf_ref[...], ewt_ref[...],
                 preferred_element_type=jnp.float32)
    wb = (wb + eb_ref[...]).astype(jnp.bfloat16).astype(jnp.float32)
    hrep = jnp.dot(hs_ref[...].astype(jnp.bfloat16), rep_ref[...],
                   preferred_element_type=jnp.float32)
    x = hrep * wb
    t = x[:, 0:128]
    for k in range(1, 8):
        t = t + x[:, k * 128:(k + 1) * 128]
    out_ref[...] = (t[:, 0:32] + t[:, 32:64]) + (t[:, 64:96] + t[:, 96:128])


def _msg(h_src, bf_p, ewt, eb, rep, e_pad):
    blk = 1024
    grid = e_pad // blk
    return pl.pallas_call(
        _msg_body,
        grid=(grid,),
        in_specs=[
            pl.BlockSpec((blk, HID), lambda i: (i, 0)),
            pl.BlockSpec((blk, 16), lambda i: (i, 0)),
            pl.BlockSpec((16, HID * HID), lambda i: (0, 0)),
            pl.BlockSpec((1, HID * HID), lambda i: (0, 0)),
            pl.BlockSpec((HID, HID * HID), lambda i: (0, 0)),
        ],
        out_specs=pl.BlockSpec((blk, HID), lambda i: (i, 0)),
        out_shape=jax.ShapeDtypeStruct((e_pad, HID), jnp.float32),
    )(h_src, bf_p, ewt, eb, rep)


def _scatter(msg, dst3, zeros, n_pad, nch):
    mesh = plsc.VectorSubcoreMesh(core_axis_name="c", subcore_axis_name="s")
    rps = n_pad // NS

    @functools.partial(
        pl.kernel,
        mesh=mesh,
        out_type=jax.ShapeDtypeStruct((NC, n_pad, HID), jnp.float32),
        compiler_params=pltpu.CompilerParams(use_tc_tiling_on_sc=False),
        scratch_types=[
            pltpu.VMEM((nch, CHUNK), jnp.int32),
            pltpu.VMEM((CHUNK, HID), jnp.float32),
            pltpu.VMEM_SHARED((n_pad, HID), jnp.float32),
        ],
    )
    def k(msg_hbm, dst_hbm, z_hbm, out_hbm, idx_v, msg_v, agg_sh):
        c = lax.axis_index("c")
        s = lax.axis_index("s")
        wid = c * NS + s
        pltpu.sync_copy(z_hbm.at[pl.ds(s * rps, rps)],
                        agg_sh.at[pl.ds(s * rps, rps)])
        pltpu.sync_copy(dst_hbm.at[wid], idx_v)
        plsc.subcore_barrier()

        def body(j, carry):
            off = (wid * nch + j) * CHUNK
            pltpu.sync_copy(msg_hbm.at[pl.ds(off, CHUNK)], msg_v)
            pltpu.sync_copy(msg_v, agg_sh.at[idx_v.at[j]], add=True)
            return carry

        lax.fori_loop(0, nch, body, 0)
        plsc.subcore_barrier()
        pltpu.sync_copy(agg_sh.at[pl.ds(s * rps, rps)],
                        out_hbm.at[c, pl.ds(s * rps, rps)])

    return k(msg, dst3, zeros)


def _gru_body(h_ref, aggp_ref, wt_ref, bias_ref, out_ref):
    h = h_ref[...]
    agg = aggp_ref[0] + aggp_ref[1]

    def mm(x, i):
        return (jnp.dot(x, wt_ref[i], preferred_element_type=jnp.float32)
                + bias_ref[i:i + 1, :])

    r = jax.nn.sigmoid(mm(agg, 0) + mm(h, 3))
    z = jax.nn.sigmoid(mm(agg, 1) + mm(h, 4))
    nn = jnp.tanh(mm(agg, 2) + r * mm(h, 5))
    out_ref[...] = (1.0 - z) * nn + z * h


def _gru(h, aggp, wt, bias):
    n = h.shape[0]
    blk = 1000
    return pl.pallas_call(
        _gru_body,
        grid=(n // blk,),
        in_specs=[
            pl.BlockSpec((blk, HID), lambda i: (i, 0)),
            pl.BlockSpec((NC, blk, HID), lambda i: (0, i, 0)),
            pl.BlockSpec((6, HID, HID), lambda i: (0, 0, 0)),
            pl.BlockSpec((6, HID), lambda i: (0, 0)),
        ],
        out_specs=pl.BlockSpec((blk, HID), lambda i: (i, 0)),
        out_shape=jax.ShapeDtypeStruct((n, HID), jnp.float32),
    )(h, aggp, wt, bias)


def kernel(af, bf, edge_index, enc1_w, enc1_b, bn1_g, bn1_b, enc2_w, enc2_b,
           bn2_g, bn2_b, ef_w, ef_b, gru_wi, gru_wh, gru_bi, gru_bh):
    n = af.shape[0]
    e = bf.shape[0]
    bond = bf.shape[1]

    nch = -(-e // (NW * CHUNK))
    e_pad = NW * nch * CHUNK
    n_pad = -(-(n + 1) // NS) * NS
    dump = n_pad - 1

    p1 = jnp.stack([enc1_b, bn1_g, bn1_b])
    p2 = jnp.stack([enc2_b, bn2_g, bn2_b])
    h = _encoder(af, enc1_w.T, p1, enc2_w.T, p2)

    src = edge_index[0].astype(jnp.int32)
    dst = edge_index[1].astype(jnp.int32)
    src3 = jnp.pad(src, (0, e_pad - e)).reshape(NW, nch, CHUNK)
    dst3 = jnp.pad(dst, (0, e_pad - e),
                   constant_values=dump).reshape(NW, nch, CHUNK)
    bf_p = jnp.pad(bf, ((0, e_pad - e), (0, 0))).astype(jnp.bfloat16)

    h_src = _gather(h, src3, e_pad, nch)

    ewt = ef_w.T.astype(jnp.bfloat16)
    eb = ef_b.reshape(1, HID * HID)
    eye = jnp.eye(HID, dtype=jnp.float32)
    rep = jnp.kron(eye, jnp.ones((1, HID), jnp.float32)).astype(jnp.bfloat16)
    msg = _msg(h_src, bf_p, ewt, eb, rep, e_pad)

    zeros = jnp.zeros((n_pad, HID), jnp.float32)
    aggp = _scatter(msg, dst3, zeros, n_pad, nch)

    wt = jnp.stack([
        gru_wi[:HID].T, gru_wi[HID:2 * HID].T, gru_wi[2 * HID:].T,
        gru_wh[:HID].T, gru_wh[HID:2 * HID].T, gru_wh[2 * HID:].T,
    ])
    bias = jnp.stack([
        gru_bi[:HID], gru_bi[HID:2 * HID], gru_bi[2 * HID:],
        gru_bh[:HID], gru_bh[HID:2 * HID], gru_bh[2 * HID:],
    ])
    return _gru(h, aggp, wt, bias)


# speedup vs baseline: 3.0747x; 1.1039x over previous
import functools

import jax
import jax.numpy as jnp
from jax import lax
from jax.experimental import pallas as pl
from jax.experimental.pallas import tpu as pltpu
from jax.experimental.pallas import tpu_sc as plsc

HID = 32
NC, NS = 2, 16
NW = NC * NS
CHUNK = 128


def _encoder_body(af_ref, w1t_ref, p1_ref, w2t_ref, p2_ref, h_ref, h16_ref):
    def layer(x, wt_ref, p_ref):
        y = jnp.dot(x, wt_ref[...], preferred_element_type=jnp.float32)
        y = y + p_ref[0:1, :]
        mu = jnp.mean(y, axis=0, keepdims=True)
        d = y - mu
        var = jnp.mean(d * d, axis=0, keepdims=True)
        y = d * (p_ref[1:2, :] * lax.rsqrt(var + 1e-5)) + p_ref[2:3, :]
        return jnp.maximum(y, 0.0)

    h = layer(af_ref[...], w1t_ref, p1_ref)
    h = layer(h, w2t_ref, p2_ref)
    h_ref[...] = h
    h16_ref[...] = h.astype(jnp.bfloat16)


def _encoder(af, w1t, p1, w2t, p2):
    n = af.shape[0]
    return pl.pallas_call(
        _encoder_body,
        out_shape=(jax.ShapeDtypeStruct((n, HID), jnp.float32),
                   jax.ShapeDtypeStruct((n, HID), jnp.bfloat16)),
    )(af, w1t, p1, w2t, p2)


GDEPTH = 8


def _gather(h16, src3, e_pad, nch):
    mesh = plsc.VectorSubcoreMesh(core_axis_name="c", subcore_axis_name="s")

    @functools.partial(
        pl.kernel,
        mesh=mesh,
        out_type=jax.ShapeDtypeStruct((e_pad, HID), jnp.bfloat16),
        compiler_params=pltpu.CompilerParams(use_tc_tiling_on_sc=False),
        scratch_types=[
            pltpu.VMEM((nch, CHUNK), jnp.int32),
            pltpu.VMEM((GDEPTH * CHUNK, HID), jnp.bfloat16),
        ] + [pltpu.SemaphoreType.DMA] * GDEPTH,
    )
    def k(h_hbm, src_hbm, out_hbm, idx_v, rows_v, *sems):
        wid = lax.axis_index("c") * NS + lax.axis_index("s")
        pltpu.sync_copy(src_hbm.at[wid], idx_v)

        def body(g, carry):
            copies = []
            for b in range(GDEPTH):
                cj = g * GDEPTH + b
                copies.append(pltpu.async_copy(
                    h_hbm.at[idx_v.at[cj]],
                    rows_v.at[pl.ds(b * CHUNK, CHUNK)], sems[b]))
            for cp in copies:
                cp.wait()
            off = (wid * nch + g * GDEPTH) * CHUNK
            pltpu.sync_copy(rows_v, out_hbm.at[pl.ds(off, GDEPTH * CHUNK)])
            return carry

        lax.fori_loop(0, nch // GDEPTH, body, 0)

    return k(h16, src3)


def _msg_body(hs_ref, bf_ref, ewt_ref, eb_ref, rep_ref, out_ref):
    wb = jnp.dot(bf_ref[...], ewt_ref[...],
                 preferred_element_type=jnp.float32)
    wb = (wb + eb_ref[...]).astype(jnp.bfloat16).astype(jnp.float32)
    hrep = jnp.dot(hs_ref[...].astype(jnp.bfloat16), rep_ref[...],
                   preferred_element_type=jnp.float32)
    x = hrep * wb
    t = x[:, 0:128]
    for k in range(1, 8):
        t = t + x[:, k * 128:(k + 1) * 128]
    out_ref[...] = (t[:, 0:32] + t[:, 32:64]) + (t[:, 64:96] + t[:, 96:128])


def _msg(h_src, bf_p, ewt, eb, rep, e_pad):
    blk = 1024
    grid = e_pad // blk
    return pl.pallas_call(
        _msg_body,
        grid=(grid,),
        in_specs=[
            pl.BlockSpec((blk, HID), lambda i: (i, 0)),
            pl.BlockSpec((blk, 16), lambda i: (i, 0)),
            pl.BlockSpec((16, HID * HID), lambda i: (0, 0)),
            pl.BlockSpec((1, HID * HID), lambda i: (0, 0)),
            pl.BlockSpec((HID, HID * HID), lambda i: (0, 0)),
        ],
        out_specs=pl.BlockSpec((blk, HID), lambda i: (i, 0)),
        out_shape=jax.ShapeDtypeStruct((e_pad, HID), jnp.float32),
    )(h_src, bf_p, ewt, eb, rep)


def _scatter(msg, dst3, zeros, n_pad, nch):
    mesh = plsc.VectorSubcoreMesh(core_axis_name="c", subcore_axis_name="s")
    rps = n_pad // NS

    @functools.partial(
        pl.kernel,
        mesh=mesh,
        out_type=jax.ShapeDtypeStruct((NC, n_pad, HID), jnp.float32),
        compiler_params=pltpu.CompilerParams(use_tc_tiling_on_sc=False),
        scratch_types=[
            pltpu.VMEM((nch, CHUNK), jnp.int32),
            pltpu.VMEM((GDEPTH * CHUNK, HID), jnp.float32),
            pltpu.VMEM_SHARED((n_pad, HID), jnp.float32),
        ],
    )
    def k(msg_hbm, dst_hbm, z_hbm, out_hbm, idx_v, msg_v, agg_sh):
        c = lax.axis_index("c")
        s = lax.axis_index("s")
        wid = c * NS + s
        pltpu.sync_copy(z_hbm.at[pl.ds(s * rps, rps)],
                        agg_sh.at[pl.ds(s * rps, rps)])
        pltpu.sync_copy(dst_hbm.at[wid], idx_v)
        plsc.subcore_barrier()

        def body(g, carry):
            off = (wid * nch + g * GDEPTH) * CHUNK
            pltpu.sync_copy(msg_hbm.at[pl.ds(off, GDEPTH * CHUNK)], msg_v)
            for b in range(GDEPTH):
                pltpu.sync_copy(msg_v.at[pl.ds(b * CHUNK, CHUNK)],
                                agg_sh.at[idx_v.at[g * GDEPTH + b]], add=True)
            return carry

        lax.fori_loop(0, nch // GDEPTH, body, 0)
        plsc.subcore_barrier()
        pltpu.sync_copy(agg_sh.at[pl.ds(s * rps, rps)],
                        out_hbm.at[c, pl.ds(s * rps, rps)])

    return k(msg, dst3, zeros)


def _gru_body(h_ref, aggp_ref, wt_ref, bias_ref, out_ref):
    h = h_ref[...]
    agg = aggp_ref[0] + aggp_ref[1]

    def mm(x, i):
        return (jnp.dot(x, wt_ref[i], preferred_element_type=jnp.float32)
                + bias_ref[i:i + 1, :])

    r = jax.nn.sigmoid(mm(agg, 0) + mm(h, 3))
    z = jax.nn.sigmoid(mm(agg, 1) + mm(h, 4))
    nn = jnp.tanh(mm(agg, 2) + r * mm(h, 5))
    out_ref[...] = (1.0 - z) * nn + z * h


def _gru(h, aggp, wt, bias):
    n = h.shape[0]
    blk = 1000
    return pl.pallas_call(
        _gru_body,
        grid=(n // blk,),
        in_specs=[
            pl.BlockSpec((blk, HID), lambda i: (i, 0)),
            pl.BlockSpec((NC, blk, HID), lambda i: (0, i, 0)),
            pl.BlockSpec((6, HID, HID), lambda i: (0, 0, 0)),
            pl.BlockSpec((6, HID), lambda i: (0, 0)),
        ],
        out_specs=pl.BlockSpec((blk, HID), lambda i: (i, 0)),
        out_shape=jax.ShapeDtypeStruct((n, HID), jnp.float32),
    )(h, aggp, wt, bias)


def kernel(af, bf, edge_index, enc1_w, enc1_b, bn1_g, bn1_b, enc2_w, enc2_b,
           bn2_g, bn2_b, ef_w, ef_b, gru_wi, gru_wh, gru_bi, gru_bh):
    n = af.shape[0]
    e = bf.shape[0]
    bond = bf.shape[1]

    nch = -(-e // (NW * CHUNK))
    nch = -(-nch // GDEPTH) * GDEPTH
    e_pad = NW * nch * CHUNK
    n_pad = -(-(n + 1) // NS) * NS
    dump = n_pad - 1

    p1 = jnp.stack([enc1_b, bn1_g, bn1_b])
    p2 = jnp.stack([enc2_b, bn2_g, bn2_b])
    h, h16 = _encoder(af, enc1_w.T, p1, enc2_w.T, p2)

    src = edge_index[0].astype(jnp.int32)
    dst = edge_index[1].astype(jnp.int32)
    src3 = jnp.pad(src, (0, e_pad - e)).reshape(NW, nch, CHUNK)
    dst3 = jnp.pad(dst, (0, e_pad - e),
                   constant_values=dump).reshape(NW, nch, CHUNK)
    bf_p = jnp.pad(bf, ((0, e_pad - e), (0, 0))).astype(jnp.bfloat16)

    h_src = _gather(h16, src3, e_pad, nch)

    ewt = ef_w.T.astype(jnp.bfloat16)
    eb = ef_b.reshape(1, HID * HID)
    eye = jnp.eye(HID, dtype=jnp.float32)
    rep = jnp.kron(eye, jnp.ones((1, HID), jnp.float32)).astype(jnp.bfloat16)
    msg = _msg(h_src, bf_p, ewt, eb, rep, e_pad)

    zeros = jnp.zeros((n_pad, HID), jnp.float32)
    aggp = _scatter(msg, dst3, zeros, n_pad, nch)

    wt = jnp.stack([
        gru_wi[:HID].T, gru_wi[HID:2 * HID].T, gru_wi[2 * HID:].T,
        gru_wh[:HID].T, gru_wh[HID:2 * HID].T, gru_wh[2 * HID:].T,
    ])
    bias = jnp.stack([
        gru_bi[:HID], gru_bi[HID:2 * HID], gru_bi[2 * HID:],
        gru_bh[:HID], gru_bh[HID:2 * HID], gru_bh[2 * HID:],
    ])
    return _gru(h, aggp, wt, bias)


# speedup vs baseline: 3.2361x; 1.0525x over previous
import functools

import jax
import jax.numpy as jnp
from jax import lax
from jax.experimental import pallas as pl
from jax.experimental.pallas import tpu as pltpu
from jax.experimental.pallas import tpu_sc as plsc

HID = 32
NC, NS = 2, 16
NW = NC * NS
CHUNK = 128


def _encoder_body(af_ref, w1t_ref, p1_ref, w2t_ref, p2_ref, h_ref, h16_ref):
    def layer(x, wt_ref, p_ref):
        y = jnp.dot(x, wt_ref[...], preferred_element_type=jnp.float32)
        y = y + p_ref[0:1, :]
        mu = jnp.mean(y, axis=0, keepdims=True)
        d = y - mu
        var = jnp.mean(d * d, axis=0, keepdims=True)
        y = d * (p_ref[1:2, :] * lax.rsqrt(var + 1e-5)) + p_ref[2:3, :]
        return jnp.maximum(y, 0.0)

    h = layer(af_ref[...], w1t_ref, p1_ref)
    h = layer(h, w2t_ref, p2_ref)
    h_ref[...] = h
    h16_ref[...] = h.astype(jnp.bfloat16)


def _encoder(af, w1t, p1, w2t, p2):
    n = af.shape[0]
    return pl.pallas_call(
        _encoder_body,
        out_shape=(jax.ShapeDtypeStruct((n, HID), jnp.float32),
                   jax.ShapeDtypeStruct((n, HID), jnp.bfloat16)),
    )(af, w1t, p1, w2t, p2)


GDEPTH = 8


def _gather(h16, src3, e_pad, nch):
    mesh = plsc.VectorSubcoreMesh(core_axis_name="c", subcore_axis_name="s")

    @functools.partial(
        pl.kernel,
        mesh=mesh,
        out_type=jax.ShapeDtypeStruct((e_pad, HID), jnp.bfloat16),
        compiler_params=pltpu.CompilerParams(use_tc_tiling_on_sc=False),
        scratch_types=[
            pltpu.VMEM((nch, CHUNK), jnp.int32),
            pltpu.VMEM((GDEPTH * CHUNK, HID), jnp.bfloat16),
        ] + [pltpu.SemaphoreType.DMA] * GDEPTH,
    )
    def k(h_hbm, src_hbm, out_hbm, idx_v, rows_v, *sems):
        wid = lax.axis_index("c") * NS + lax.axis_index("s")
        pltpu.sync_copy(src_hbm.at[wid], idx_v)

        def body(g, carry):
            copies = []
            for b in range(GDEPTH):
                cj = g * GDEPTH + b
                copies.append(pltpu.async_copy(
                    h_hbm.at[idx_v.at[cj]],
                    rows_v.at[pl.ds(b * CHUNK, CHUNK)], sems[b]))
            for cp in copies:
                cp.wait()
            off = (wid * nch + g * GDEPTH) * CHUNK
            pltpu.sync_copy(rows_v, out_hbm.at[pl.ds(off, GDEPTH * CHUNK)])
            return carry

        lax.fori_loop(0, nch // GDEPTH, body, 0)

    return k(h16, src3)


def _msg_body(hs_ref, bf_ref, ewt_ref, eb_ref, rep_ref, out_ref):
    wb = jnp.dot(bf_ref[...], ewt_ref[...],
                 preferred_element_type=jnp.float32)
    wb = (wb + eb_ref[...]).astype(jnp.bfloat16).astype(jnp.float32)
    hrep = jnp.dot(hs_ref[...].astype(jnp.bfloat16), rep_ref[...],
                   preferred_element_type=jnp.float32)
    x = hrep * wb
    t = x[:, 0:128]
    for k in range(1, 8):
        t = t + x[:, k * 128:(k + 1) * 128]
    out_ref[...] = (t[:, 0:32] + t[:, 32:64]) + (t[:, 64:96] + t[:, 96:128])


def _msg(h_src, bf_p, ewt, eb, rep, e_pad):
    blk = 2048
    grid = e_pad // blk
    return pl.pallas_call(
        _msg_body,
        grid=(grid,),
        in_specs=[
            pl.BlockSpec((blk, HID), lambda i: (i, 0)),
            pl.BlockSpec((blk, 16), lambda i: (i, 0)),
            pl.BlockSpec((16, HID * HID), lambda i: (0, 0)),
            pl.BlockSpec((1, HID * HID), lambda i: (0, 0)),
            pl.BlockSpec((HID, HID * HID), lambda i: (0, 0)),
        ],
        out_specs=pl.BlockSpec((blk, HID), lambda i: (i, 0)),
        out_shape=jax.ShapeDtypeStruct((e_pad, HID), jnp.float32),
    )(h_src, bf_p, ewt, eb, rep)


def _scatter(msg, dst3, zeros, n_pad, nch):
    mesh = plsc.VectorSubcoreMesh(core_axis_name="c", subcore_axis_name="s")
    rps = n_pad // NS

    @functools.partial(
        pl.kernel,
        mesh=mesh,
        out_type=jax.ShapeDtypeStruct((NC, n_pad, HID), jnp.float32),
        compiler_params=pltpu.CompilerParams(use_tc_tiling_on_sc=False),
        scratch_types=[
            pltpu.VMEM((nch, CHUNK), jnp.int32),
            pltpu.VMEM((GDEPTH * CHUNK, HID), jnp.float32),
            pltpu.VMEM_SHARED((n_pad, HID), jnp.float32),
        ],
    )
    def k(msg_hbm, dst_hbm, z_hbm, out_hbm, idx_v, msg_v, agg_sh):
        c = lax.axis_index("c")
        s = lax.axis_index("s")
        wid = c * NS + s
        pltpu.sync_copy(z_hbm.at[pl.ds(s * rps, rps)],
                        agg_sh.at[pl.ds(s * rps, rps)])
        pltpu.sync_copy(dst_hbm.at[wid], idx_v)
        plsc.subcore_barrier()

        def body(g, carry):
            off = (wid * nch + g * GDEPTH) * CHUNK
            pltpu.sync_copy(msg_hbm.at[pl.ds(off, GDEPTH * CHUNK)], msg_v)
            for b in range(GDEPTH):
                pltpu.sync_copy(msg_v.at[pl.ds(b * CHUNK, CHUNK)],
                                agg_sh.at[idx_v.at[g * GDEPTH + b]], add=True)
            return carry

        lax.fori_loop(0, nch // GDEPTH, body, 0)
        plsc.subcore_barrier()
        pltpu.sync_copy(agg_sh.at[pl.ds(s * rps, rps)],
                        out_hbm.at[c, pl.ds(s * rps, rps)])

    return k(msg, dst3, zeros)


def _gru_body(h_ref, aggp_ref, wt_ref, bias_ref, out_ref):
    h = h_ref[...]
    agg = aggp_ref[0] + aggp_ref[1]

    def mm(x, i):
        return (jnp.dot(x, wt_ref[i], preferred_element_type=jnp.float32)
                + bias_ref[i:i + 1, :])

    r = jax.nn.sigmoid(mm(agg, 0) + mm(h, 3))
    z = jax.nn.sigmoid(mm(agg, 1) + mm(h, 4))
    nn = jnp.tanh(mm(agg, 2) + r * mm(h, 5))
    out_ref[...] = (1.0 - z) * nn + z * h


def _gru(h, aggp, wt, bias):
    n = h.shape[0]
    blk = 1000
    return pl.pallas_call(
        _gru_body,
        grid=(n // blk,),
        in_specs=[
            pl.BlockSpec((blk, HID), lambda i: (i, 0)),
            pl.BlockSpec((NC, blk, HID), lambda i: (0, i, 0)),
            pl.BlockSpec((6, HID, HID), lambda i: (0, 0, 0)),
            pl.BlockSpec((6, HID), lambda i: (0, 0)),
        ],
        out_specs=pl.BlockSpec((blk, HID), lambda i: (i, 0)),
        out_shape=jax.ShapeDtypeStruct((n, HID), jnp.float32),
    )(h, aggp, wt, bias)


def kernel(af, bf, edge_index, enc1_w, enc1_b, bn1_g, bn1_b, enc2_w, enc2_b,
           bn2_g, bn2_b, ef_w, ef_b, gru_wi, gru_wh, gru_bi, gru_bh):
    n = af.shape[0]
    e = bf.shape[0]
    bond = bf.shape[1]

    nch = -(-e // (NW * CHUNK))
    nch = -(-nch // GDEPTH) * GDEPTH
    e_pad = NW * nch * CHUNK
    n_pad = -(-(n + 1) // NS) * NS
    dump = n_pad - 1

    p1 = jnp.stack([enc1_b, bn1_g, bn1_b])
    p2 = jnp.stack([enc2_b, bn2_g, bn2_b])
    h, h16 = _encoder(af, enc1_w.T, p1, enc2_w.T, p2)

    src = edge_index[0].astype(jnp.int32)
    dst = edge_index[1].astype(jnp.int32)
    src3 = jnp.pad(src, (0, e_pad - e)).reshape(NW, nch, CHUNK)
    dst3 = jnp.pad(dst, (0, e_pad - e),
                   constant_values=dump).reshape(NW, nch, CHUNK)
    bf_p = jnp.pad(bf, ((0, e_pad - e), (0, 0))).astype(jnp.bfloat16)

    h_src = _gather(h16, src3, e_pad, nch)

    ewt = ef_w.T.astype(jnp.bfloat16)
    eb = ef_b.reshape(1, HID * HID)
    eye = jnp.eye(HID, dtype=jnp.float32)
    rep = jnp.kron(eye, jnp.ones((1, HID), jnp.float32)).astype(jnp.bfloat16)
    msg = _msg(h_src, bf_p, ewt, eb, rep, e_pad)

    zeros = jnp.zeros((n_pad, HID), jnp.float32)
    aggp = _scatter(msg, dst3, zeros, n_pad, nch)

    wt = jnp.stack([
        gru_wi[:HID].T, gru_wi[HID:2 * HID].T, gru_wi[2 * HID:].T,
        gru_wh[:HID].T, gru_wh[HID:2 * HID].T, gru_wh[2 * HID:].T,
    ])
    bias = jnp.stack([
        gru_bi[:HID], gru_bi[HID:2 * HID], gru_bi[2 * HID:],
        gru_bh[:HID], gru_bh[HID:2 * HID], gru_bh[2 * HID:],
    ])
    return _gru(h, aggp, wt, bias)


# speedup vs baseline: 3.3140x; 1.0241x over previous
import functools

import jax
import jax.numpy as jnp
from jax import lax
from jax.experimental import pallas as pl
from jax.experimental.pallas import tpu as pltpu
from jax.experimental.pallas import tpu_sc as plsc

HID = 32
NC, NS = 2, 16
NW = NC * NS
CHUNK = 128


def _encoder_body(af_ref, w1t_ref, p1_ref, w2t_ref, p2_ref, h_ref, h16_ref):
    def layer(x, wt_ref, p_ref):
        y = jnp.dot(x, wt_ref[...], preferred_element_type=jnp.float32)
        y = y + p_ref[0:1, :]
        mu = jnp.mean(y, axis=0, keepdims=True)
        d = y - mu
        var = jnp.mean(d * d, axis=0, keepdims=True)
        y = d * (p_ref[1:2, :] * lax.rsqrt(var + 1e-5)) + p_ref[2:3, :]
        return jnp.maximum(y, 0.0)

    h = layer(af_ref[...], w1t_ref, p1_ref)
    h = layer(h, w2t_ref, p2_ref)
    h_ref[...] = h
    h16_ref[...] = h.astype(jnp.bfloat16)


def _encoder(af, w1t, p1, w2t, p2):
    n = af.shape[0]
    return pl.pallas_call(
        _encoder_body,
        out_shape=(jax.ShapeDtypeStruct((n, HID), jnp.float32),
                   jax.ShapeDtypeStruct((n, HID), jnp.bfloat16)),
    )(af, w1t, p1, w2t, p2)


GDEPTH = 8


def _gather(h16, src3, e_pad, nch):
    mesh = plsc.VectorSubcoreMesh(core_axis_name="c", subcore_axis_name="s")

    @functools.partial(
        pl.kernel,
        mesh=mesh,
        out_type=jax.ShapeDtypeStruct((e_pad, HID), jnp.bfloat16),
        compiler_params=pltpu.CompilerParams(use_tc_tiling_on_sc=False),
        scratch_types=[
            pltpu.VMEM((nch, CHUNK), jnp.int32),
            pltpu.VMEM((GDEPTH * CHUNK, HID), jnp.bfloat16),
        ] + [pltpu.SemaphoreType.DMA] * GDEPTH,
    )
    def k(h_hbm, src_hbm, out_hbm, idx_v, rows_v, *sems):
        wid = lax.axis_index("c") * NS + lax.axis_index("s")
        pltpu.sync_copy(src_hbm.at[wid], idx_v)

        def body(g, carry):
            copies = []
            for b in range(GDEPTH):
                cj = g * GDEPTH + b
                copies.append(pltpu.async_copy(
                    h_hbm.at[idx_v.at[cj]],
                    rows_v.at[pl.ds(b * CHUNK, CHUNK)], sems[b]))
            for cp in copies:
                cp.wait()
            off = (wid * nch + g * GDEPTH) * CHUNK
            pltpu.sync_copy(rows_v, out_hbm.at[pl.ds(off, GDEPTH * CHUNK)])
            return carry

        lax.fori_loop(0, nch // GDEPTH, body, 0)

    return k(h16, src3)


def _msg_body(hs_ref, bf_ref, ewt_ref, eb_ref, rep_ref, out_ref):
    wb = jnp.dot(bf_ref[...], ewt_ref[...],
                 preferred_element_type=jnp.float32)
    wb = (wb + eb_ref[...]).astype(jnp.bfloat16).astype(jnp.float32)
    hrep = jnp.dot(hs_ref[...].astype(jnp.bfloat16), rep_ref[...],
                   preferred_element_type=jnp.float32)
    x = hrep * wb
    t = x[:, 0:128]
    for k in range(1, 8):
        t = t + x[:, k * 128:(k + 1) * 128]
    out_ref[...] = (t[:, 0:32] + t[:, 32:64]) + (t[:, 64:96] + t[:, 96:128])


def _msg(h_src, bf_p, ewt, eb, rep, e_pad):
    blk = 4096
    grid = e_pad // blk
    return pl.pallas_call(
        _msg_body,
        grid=(grid,),
        in_specs=[
            pl.BlockSpec((blk, HID), lambda i: (i, 0)),
            pl.BlockSpec((blk, 16), lambda i: (i, 0)),
            pl.BlockSpec((16, HID * HID), lambda i: (0, 0)),
            pl.BlockSpec((1, HID * HID), lambda i: (0, 0)),
            pl.BlockSpec((HID, HID * HID), lambda i: (0, 0)),
        ],
        out_specs=pl.BlockSpec((blk, HID), lambda i: (i, 0)),
        out_shape=jax.ShapeDtypeStruct((e_pad, HID), jnp.float32),
    )(h_src, bf_p, ewt, eb, rep)


def _scatter(msg, dst3, zeros, n_pad, nch):
    mesh = plsc.VectorSubcoreMesh(core_axis_name="c", subcore_axis_name="s")
    rps = n_pad // NS

    @functools.partial(
        pl.kernel,
        mesh=mesh,
        out_type=jax.ShapeDtypeStruct((NC, n_pad, HID), jnp.float32),
        compiler_params=pltpu.CompilerParams(use_tc_tiling_on_sc=False),
        scratch_types=[
            pltpu.VMEM((nch, CHUNK), jnp.int32),
            pltpu.VMEM((GDEPTH * CHUNK, HID), jnp.float32),
            pltpu.VMEM_SHARED((n_pad, HID), jnp.float32),
        ],
    )
    def k(msg_hbm, dst_hbm, z_hbm, out_hbm, idx_v, msg_v, agg_sh):
        c = lax.axis_index("c")
        s = lax.axis_index("s")
        wid = c * NS + s
        pltpu.sync_copy(z_hbm.at[pl.ds(s * rps, rps)],
                        agg_sh.at[pl.ds(s * rps, rps)])
        pltpu.sync_copy(dst_hbm.at[wid], idx_v)
        plsc.subcore_barrier()

        def body(g, carry):
            off = (wid * nch + g * GDEPTH) * CHUNK
            pltpu.sync_copy(msg_hbm.at[pl.ds(off, GDEPTH * CHUNK)], msg_v)
            for b in range(GDEPTH):
                pltpu.sync_copy(msg_v.at[pl.ds(b * CHUNK, CHUNK)],
                                agg_sh.at[idx_v.at[g * GDEPTH + b]], add=True)
            return carry

        lax.fori_loop(0, nch // GDEPTH, body, 0)
        plsc.subcore_barrier()
        pltpu.sync_copy(agg_sh.at[pl.ds(s * rps, rps)],
                        out_hbm.at[c, pl.ds(s * rps, rps)])

    return k(msg, dst3, zeros)


def _gru_body(h_ref, aggp_ref, wt_ref, bias_ref, out_ref):
    h = h_ref[...]
    agg = aggp_ref[0] + aggp_ref[1]

    def mm(x, i):
        return (jnp.dot(x, wt_ref[i], preferred_element_type=jnp.float32)
                + bias_ref[i:i + 1, :])

    r = jax.nn.sigmoid(mm(agg, 0) + mm(h, 3))
    z = jax.nn.sigmoid(mm(agg, 1) + mm(h, 4))
    nn = jnp.tanh(mm(agg, 2) + r * mm(h, 5))
    out_ref[...] = (1.0 - z) * nn + z * h


def _gru(h, aggp, wt, bias):
    n = h.shape[0]
    blk = 1000
    return pl.pallas_call(
        _gru_body,
        grid=(n // blk,),
        in_specs=[
            pl.BlockSpec((blk, HID), lambda i: (i, 0)),
            pl.BlockSpec((NC, blk, HID), lambda i: (0, i, 0)),
            pl.BlockSpec((6, HID, HID), lambda i: (0, 0, 0)),
            pl.BlockSpec((6, HID), lambda i: (0, 0)),
        ],
        out_specs=pl.BlockSpec((blk, HID), lambda i: (i, 0)),
        out_shape=jax.ShapeDtypeStruct((n, HID), jnp.float32),
    )(h, aggp, wt, bias)


def kernel(af, bf, edge_index, enc1_w, enc1_b, bn1_g, bn1_b, enc2_w, enc2_b,
           bn2_g, bn2_b, ef_w, ef_b, gru_wi, gru_wh, gru_bi, gru_bh):
    n = af.shape[0]
    e = bf.shape[0]
    bond = bf.shape[1]

    nch = -(-e // (NW * CHUNK))
    nch = -(-nch // GDEPTH) * GDEPTH
    e_pad = NW * nch * CHUNK
    n_pad = -(-(n + 1) // NS) * NS
    dump = n_pad - 1

    p1 = jnp.stack([enc1_b, bn1_g, bn1_b])
    p2 = jnp.stack([enc2_b, bn2_g, bn2_b])
    h, h16 = _encoder(af, enc1_w.T, p1, enc2_w.T, p2)

    src = edge_index[0].astype(jnp.int32)
    dst = edge_index[1].astype(jnp.int32)
    src3 = jnp.pad(src, (0, e_pad - e)).reshape(NW, nch, CHUNK)
    dst3 = jnp.pad(dst, (0, e_pad - e),
                   constant_values=dump).reshape(NW, nch, CHUNK)
    bf_p = jnp.pad(bf, ((0, e_pad - e), (0, 0))).astype(jnp.bfloat16)

    h_src = _gather(h16, src3, e_pad, nch)

    ewt = ef_w.T.astype(jnp.bfloat16)
    eb = ef_b.reshape(1, HID * HID)
    eye = jnp.eye(HID, dtype=jnp.float32)
    rep = jnp.kron(eye, jnp.ones((1, HID), jnp.float32)).astype(jnp.bfloat16)
    msg = _msg(h_src, bf_p, ewt, eb, rep, e_pad)

    zeros = jnp.zeros((n_pad, HID), jnp.float32)
    aggp = _scatter(msg, dst3, zeros, n_pad, nch)

    wt = jnp.stack([
        gru_wi[:HID].T, gru_wi[HID:2 * HID].T, gru_wi[2 * HID:].T,
        gru_wh[:HID].T, gru_wh[HID:2 * HID].T, gru_wh[2 * HID:].T,
    ])
    bias = jnp.stack([
        gru_bi[:HID], gru_bi[HID:2 * HID], gru_bi[2 * HID:],
        gru_bh[:HID], gru_bh[HID:2 * HID], gru_bh[2 * HID:],
    ])
    return _gru(h, aggp, wt, bias)
